# Initial kernel scaffold; baseline (speedup 1.0000x reference)
import jax
import jax.numpy as jnp
from jax.experimental import pallas as pl


def kernel(conditions, table):
    raise NotImplementedError("write your pallas kernel here")


# speedup vs baseline: 2.5033x; 2.5033x over previous
import functools

import jax
import jax.numpy as jnp
from jax import lax
from jax.experimental import pallas as pl
from jax.experimental.pallas import tpu as pltpu
from jax.experimental.pallas import tpu_sc as plsc

B, L, DIM = 16384, 200, 16
N = B * L
NC, NS = 2, 16
NW = NC * NS
PER_W = N // NW
CHUNK = 2048
N_CHUNKS = PER_W // CHUNK


def _sc_gather(idx_flat, table):
    mesh = plsc.VectorSubcoreMesh(core_axis_name="c", subcore_axis_name="s")

    @functools.partial(
        pl.kernel,
        mesh=mesh,
        out_type=jax.ShapeDtypeStruct((N, DIM), jnp.float32),
        scratch_types=[
            pltpu.VMEM((CHUNK,), jnp.int32),
            pltpu.VMEM((CHUNK, DIM), jnp.float32),
            pltpu.SemaphoreType.DMA,
        ],
        compiler_params=pltpu.CompilerParams(use_tc_tiling_on_sc=False),
    )
    def k(table_hbm, idx_hbm, out_hbm, idx_v, rows_v, sem):
        wid = lax.axis_index("s") * NC + lax.axis_index("c")
        base = wid * PER_W

        def body(g, carry):
            off = base + g * CHUNK
            pltpu.sync_copy(idx_hbm.at[pl.ds(off, CHUNK)], idx_v)
            pltpu.async_copy(table_hbm.at[idx_v], rows_v, sem).wait()
            pltpu.sync_copy(rows_v, out_hbm.at[pl.ds(off, CHUNK)])
            return carry

        lax.fori_loop(0, N_CHUNKS, body, 0)

    return k(table, idx_flat)


def kernel(conditions, table):
    idx_flat = conditions.reshape(N)
    out = _sc_gather(idx_flat, table)
    return out.reshape(B, L, DIM)


# speedup vs baseline: 2.5866x; 1.0333x over previous
import functools

import jax
import jax.numpy as jnp
from jax import lax
from jax.experimental import pallas as pl
from jax.experimental.pallas import tpu as pltpu
from jax.experimental.pallas import tpu_sc as plsc

B, L, DIM = 16384, 200, 16
N = B * L
NC, NS = 2, 16
NW = NC * NS
PER_W = N // NW
CHUNK = 2048
N_CHUNKS = PER_W // CHUNK


def _sc_gather(idx_flat, table):
    mesh = plsc.VectorSubcoreMesh(core_axis_name="c", subcore_axis_name="s")

    @functools.partial(
        pl.kernel,
        mesh=mesh,
        out_type=jax.ShapeDtypeStruct((N, DIM), jnp.float32),
        scratch_types=[
            pltpu.VMEM((CHUNK,), jnp.int32),
            pltpu.VMEM((CHUNK,), jnp.int32),
            pltpu.VMEM((CHUNK, DIM), jnp.float32),
            pltpu.VMEM((CHUNK, DIM), jnp.float32),
            pltpu.SemaphoreType.DMA,
            pltpu.SemaphoreType.DMA,
            pltpu.SemaphoreType.DMA,
            pltpu.SemaphoreType.DMA,
            pltpu.SemaphoreType.DMA,
            pltpu.SemaphoreType.DMA,
        ],
        compiler_params=pltpu.CompilerParams(use_tc_tiling_on_sc=False),
    )
    def k(table_hbm, idx_hbm, out_hbm,
          idx0, idx1, rows0, rows1,
          isem0, isem1, gsem0, gsem1, ssem0, ssem1):
        wid = lax.axis_index("s") * NC + lax.axis_index("c")
        base = wid * PER_W

        def start_load(g, idx_b, isem_b):
            off = base + jnp.minimum(g, N_CHUNKS - 1) * CHUNK
            pltpu.async_copy(idx_hbm.at[pl.ds(off, CHUNK)], idx_b, isem_b)

        def wait_load(idx_b, isem_b):
            pltpu.make_async_copy(
                idx_hbm.at[pl.ds(base, CHUNK)], idx_b, isem_b).wait()

        def start_gather(idx_b, rows_b, gsem_b):
            pltpu.async_copy(table_hbm.at[idx_b], rows_b, gsem_b)

        def wait_gather(idx_b, rows_b, gsem_b):
            pltpu.make_async_copy(
                table_hbm.at[idx_b], rows_b, gsem_b).wait()

        def start_store(g, rows_b, ssem_b):
            off = base + g * CHUNK
            pltpu.async_copy(rows_b, out_hbm.at[pl.ds(off, CHUNK)], ssem_b)

        def wait_store(rows_b, ssem_b):
            pltpu.make_async_copy(
                rows_b, out_hbm.at[pl.ds(base, CHUNK)], ssem_b).wait()

        start_load(0, idx0, isem0)
        start_load(1, idx1, isem1)
        wait_load(idx0, isem0)
        start_gather(idx0, rows0, gsem0)
        wait_load(idx1, isem1)
        start_gather(idx1, rows1, gsem1)
        wait_gather(idx0, rows0, gsem0)
        start_store(0, rows0, ssem0)
        start_load(2, idx0, isem0)

        def pair(p, carry):
            g0 = 2 * p
            wait_store(rows0, ssem0)
            wait_load(idx0, isem0)
            start_gather(idx0, rows0, gsem0)
            wait_gather(idx1, rows1, gsem1)
            start_store(g0 - 1, rows1, ssem1)
            start_load(g0 + 1, idx1, isem1)
            wait_store(rows1, ssem1)
            wait_load(idx1, isem1)
            start_gather(idx1, rows1, gsem1)
            wait_gather(idx0, rows0, gsem0)
            start_store(g0, rows0, ssem0)
            start_load(g0 + 2, idx0, isem0)
            return carry

        lax.fori_loop(1, N_CHUNKS // 2, pair, 0)

        wait_gather(idx1, rows1, gsem1)
        start_store(N_CHUNKS - 1, rows1, ssem1)
        wait_store(rows0, ssem0)
        wait_store(rows1, ssem1)
        wait_load(idx0, isem0)

    return k(table, idx_flat)


def kernel(conditions, table):
    idx_flat = conditions.reshape(N)
    out = _sc_gather(idx_flat, table)
    return out.reshape(B, L, DIM)


# speedup vs baseline: 4.9464x; 1.9123x over previous
import functools

import jax
import jax.numpy as jnp
from jax import lax
from jax.experimental import pallas as pl
from jax.experimental.pallas import tpu as pltpu
from jax.experimental.pallas import tpu_sc as plsc

B, L, DIM = 16384, 200, 16
NC, NS = 2, 16
NW = NC * NS
LO, BT = L // 8, B // 128
NBLK = LO * BT
PER_W = NBLK // NW


def _sc_embed(idx4, table):
    mesh = plsc.VectorSubcoreMesh(core_axis_name="c", subcore_axis_name="s")

    @functools.partial(
        pl.kernel,
        mesh=mesh,
        out_type=jax.ShapeDtypeStruct((L, 2, BT, 1024), jnp.float32),
        scratch_types=[
            pltpu.VMEM((8, 128), jnp.int32),
            pltpu.VMEM((8, 128), jnp.int32),
            pltpu.SemaphoreType.DMA,
            pltpu.SemaphoreType.DMA,
            pltpu.SemaphoreType.DMA,
            pltpu.SemaphoreType.DMA,
            pltpu.SemaphoreType.DMA,
            pltpu.SemaphoreType.DMA,
        ],
        compiler_params=pltpu.CompilerParams(use_tc_tiling_on_sc=False,
                                             needs_layout_passes=False),
    )
    def k(table_hbm, idx_hbm, out_hbm,
          idxA, idxB,
          isemA, isemB, gsemA, gsemB, tsemA, tsemB):
      def scoped(rowsA, rowsB, trA, trB):
        wid = lax.axis_index("s") * NC + lax.axis_index("c")
        t0 = wid * PER_W
        tlast = t0 + PER_W - 1

        def start_idx(t, idx_b, isem_b):
            t = jnp.minimum(t, tlast)
            lo = t // BT
            bt = t - lo * BT
            pltpu.async_copy(idx_hbm.at[lo, bt], idx_b, isem_b)

        def wait_idx(idx_b, isem_b):
            pltpu.make_async_copy(idx_hbm.at[0, 0], idx_b, isem_b).wait()

        def start_gathers(idx_b, rows_b, gsem_b):
            for li in range(8):
                pltpu.async_copy(table_hbm.at[idx_b.at[li]],
                                 rows_b.at[pl.ds(li * 128, 128)], gsem_b)

        def wait_gathers(idx_b, rows_b, gsem_b):
            for li in range(8):
                pltpu.make_async_copy(table_hbm.at[idx_b.at[0]],
                                      rows_b.at[pl.ds(0, 128)], gsem_b).wait()

        def wait_tr_store(tr_b, tsem_b):
            for dt in range(2):
                pltpu.make_async_copy(tr_b.at[pl.ds(dt * 1024, 1024)],
                                      out_hbm.at[0, dt, 0], tsem_b).wait()

        def transpose_store_block(t, rows_b):
            lo = t // BT
            bt = t - lo * BT

            def tpair(q, carry):
                for half in range(2):
                    tr_b, tsem_b = (trA, tsemA) if half == 0 else (trB, tsemB)

                    @pl.when(q > 0)
                    def _():
                        wait_tr_store(tr_b, tsem_b)

                    li = 2 * q + half
                    col0 = lax.iota(jnp.int32, 16) * 128
                    for jj in range(128):
                        v = rows_b[li * 128 + jj, :]
                        plsc.store_scatter(tr_b, [col0 + jj], v)
                    l_out = lo * 8 + li
                    for dt in range(2):
                        pltpu.async_copy(tr_b.at[pl.ds(dt * 1024, 1024)],
                                         out_hbm.at[l_out, dt, bt], tsem_b)
                return carry

            lax.fori_loop(0, 4, tpair, 0)
            wait_tr_store(trA, tsemA)
            wait_tr_store(trB, tsemB)

        start_idx(t0, idxA, isemA)
        wait_idx(idxA, isemA)
        start_gathers(idxA, rowsA, gsemA)
        start_idx(t0 + 1, idxB, isemB)

        def pair(p, carry):
            t = t0 + 2 * p
            wait_gathers(idxA, rowsA, gsemA)
            wait_idx(idxB, isemB)
            start_gathers(idxB, rowsB, gsemB)
            start_idx(t + 2, idxA, isemA)
            transpose_store_block(t, rowsA)
            wait_gathers(idxB, rowsB, gsemB)
            wait_idx(idxA, isemA)
            start_gathers(idxA, rowsA, gsemA)
            start_idx(t + 3, idxB, isemB)
            transpose_store_block(t + 1, rowsB)
            return carry

        lax.fori_loop(0, PER_W // 2, pair, 0)

        wait_gathers(idxA, rowsA, gsemA)
        wait_idx(idxB, isemB)

      pl.run_scoped(
          scoped,
          pltpu.VMEM((1024, DIM), jnp.float32),
          pltpu.VMEM((1024, DIM), jnp.float32),
          pltpu.VMEM((2048,), jnp.float32),
          pltpu.VMEM((2048,), jnp.float32),
      )

    return k(table, idx4)


def kernel(conditions, table):
    idx4 = conditions.reshape(128, 128, 25, 8).transpose(2, 0, 3, 1)
    out5 = _sc_embed(idx4, table)
    out5 = out5.reshape(L, 2, BT, 8, 128)
    return out5.transpose(2, 4, 0, 1, 3).reshape(B, L, DIM)


# speedup vs baseline: 7.8904x; 1.5952x over previous
import functools

import jax
import jax.numpy as jnp
from jax import lax
from jax.experimental import pallas as pl
from jax.experimental.pallas import tpu as pltpu
from jax.experimental.pallas import tpu_sc as plsc

B, L, DIM = 16384, 200, 16
NC, NS = 2, 16
NW = NC * NS
LO, BT = L // 8, B // 128
NBLK = LO * BT
PER_W = NBLK // NW


def _sc_embed(idx4, table):
    mesh = plsc.VectorSubcoreMesh(core_axis_name="c", subcore_axis_name="s")

    @functools.partial(
        pl.kernel,
        mesh=mesh,
        out_type=jax.ShapeDtypeStruct((L, 2, BT, 8, 128), jnp.float32),
        scratch_types=[
            pltpu.VMEM((8, 128), jnp.int32),
            pltpu.VMEM((8, 128), jnp.int32),
            pltpu.SemaphoreType.DMA,
            pltpu.SemaphoreType.DMA,
            pltpu.SemaphoreType.DMA,
            pltpu.SemaphoreType.DMA,
            pltpu.SemaphoreType.DMA,
            pltpu.SemaphoreType.DMA,
        ],
        compiler_params=pltpu.CompilerParams(use_tc_tiling_on_sc=False,
                                             needs_layout_passes=False),
    )
    def k(table_hbm, idx_hbm, out_hbm,
          idxA, idxB,
          isemA, isemB, gsemA, gsemB, tsemA, tsemB):
      def scoped(rowsA, rowsB, trA, trB):
        wid = lax.axis_index("s") * NC + lax.axis_index("c")
        t0 = wid * PER_W
        tlast = t0 + PER_W - 1

        def start_idx(t, idx_b, isem_b):
            t = jnp.minimum(t, tlast)
            lo = t // BT
            bt = t - lo * BT
            pltpu.async_copy(idx_hbm.at[lo, bt], idx_b, isem_b)

        def wait_idx(idx_b, isem_b):
            pltpu.make_async_copy(idx_hbm.at[0, 0], idx_b, isem_b).wait()

        def start_gathers(idx_b, rows_b, gsem_b):
            for li in range(8):
                pltpu.async_copy(table_hbm.at[idx_b.at[li]],
                                 rows_b.at[pl.ds(li * 128, 128)], gsem_b)

        def wait_gathers(idx_b, rows_b, gsem_b):
            for li in range(8):
                pltpu.make_async_copy(table_hbm.at[idx_b.at[0]],
                                      rows_b.at[pl.ds(0, 128)], gsem_b).wait()

        def wait_tr_store(tr_b, tsem_b):
            for dt in range(2):
                pltpu.make_async_copy(
                    tr_b.at[pl.ds(dt * 8, 8), pl.ds(0, 128)],
                    out_hbm.at[0, dt, 0], tsem_b).wait()

        def transpose_store_block(t, rows_b):
            lo = t // BT
            bt = t - lo * BT

            def tpair(q, carry):
                for half in range(2):
                    tr_b, tsem_b = (trA, tsemA) if half == 0 else (trB, tsemB)

                    @pl.when(q > 0)
                    def _():
                        wait_tr_store(tr_b, tsem_b)

                    li = 2 * q + half
                    rows16 = lax.iota(jnp.int32, 16)
                    for jj in range(128):
                        v = rows_b[li * 128 + jj, :]
                        col = jnp.full((16,), jj, jnp.int32)
                        plsc.store_scatter(tr_b, [rows16, col], v)
                    l_out = lo * 8 + li
                    for dt in range(2):
                        pltpu.async_copy(
                            tr_b.at[pl.ds(dt * 8, 8), pl.ds(0, 128)],
                            out_hbm.at[l_out, dt, bt], tsem_b)
                return carry

            lax.fori_loop(0, 4, tpair, 0)
            wait_tr_store(trA, tsemA)
            wait_tr_store(trB, tsemB)

        start_idx(t0, idxA, isemA)
        wait_idx(idxA, isemA)
        start_gathers(idxA, rowsA, gsemA)
        start_idx(t0 + 1, idxB, isemB)

        def pair(p, carry):
            t = t0 + 2 * p
            wait_gathers(idxA, rowsA, gsemA)
            wait_idx(idxB, isemB)
            start_gathers(idxB, rowsB, gsemB)
            start_idx(t + 2, idxA, isemA)
            transpose_store_block(t, rowsA)
            wait_gathers(idxB, rowsB, gsemB)
            wait_idx(idxA, isemA)
            start_gathers(idxA, rowsA, gsemA)
            start_idx(t + 3, idxB, isemB)
            transpose_store_block(t + 1, rowsB)
            return carry

        lax.fori_loop(0, PER_W // 2, pair, 0)

        wait_gathers(idxA, rowsA, gsemA)
        wait_idx(idxB, isemB)

      pl.run_scoped(
          scoped,
          pltpu.VMEM((1024, DIM), jnp.float32),
          pltpu.VMEM((1024, DIM), jnp.float32),
          pltpu.VMEM((16, 129), jnp.float32),
          pltpu.VMEM((16, 129), jnp.float32),
      )

    return k(table, idx4)


def kernel(conditions, table):
    idx4 = conditions.reshape(128, 128, 25, 8).transpose(2, 0, 3, 1)
    out5 = _sc_embed(idx4, table)
    out5 = out5.reshape(L, 2, BT, 8, 128)
    return out5.transpose(2, 4, 0, 1, 3).reshape(B, L, DIM)


# speedup vs baseline: 7.9283x; 1.0048x over previous
import functools

import jax
import jax.numpy as jnp
from jax import lax
from jax.experimental import pallas as pl
from jax.experimental.pallas import tpu as pltpu
from jax.experimental.pallas import tpu_sc as plsc

B, L, DIM = 16384, 200, 16
NC, NS = 2, 16
NW = NC * NS
LO, BT = L // 8, B // 128
NBLK = LO * BT
PER_W = NBLK // NW


def _sc_embed(idx4, table):
    mesh = plsc.VectorSubcoreMesh(core_axis_name="c", subcore_axis_name="s")

    @functools.partial(
        pl.kernel,
        mesh=mesh,
        out_type=jax.ShapeDtypeStruct((L, 2, BT, 8, 128), jnp.float32),
        scratch_types=[
            pltpu.VMEM((1024,), jnp.int32),
            pltpu.VMEM((1024,), jnp.int32),
            pltpu.SemaphoreType.DMA,
            pltpu.SemaphoreType.DMA,
            pltpu.SemaphoreType.DMA,
            pltpu.SemaphoreType.DMA,
            pltpu.SemaphoreType.DMA,
            pltpu.SemaphoreType.DMA,
        ],
        compiler_params=pltpu.CompilerParams(use_tc_tiling_on_sc=False,
                                             needs_layout_passes=False),
    )
    def k(table_hbm, idx_hbm, out_hbm,
          idxA, idxB,
          isemA, isemB, gsemA, gsemB, tsemA, tsemB):
      def scoped(rowsA, rowsB, trA, trB):
        wid = lax.axis_index("s") * NC + lax.axis_index("c")
        rows16 = lax.iota(jnp.int32, 16)
        zvec = jnp.zeros((16,), jnp.int32)
        t0 = wid * PER_W
        tlast = t0 + PER_W - 1

        def start_idx(t, idx_b, isem_b):
            t = jnp.minimum(t, tlast)
            lo = t // BT
            bt = t - lo * BT
            pltpu.async_copy(idx_hbm.at[lo, bt], idx_b, isem_b)

        def wait_idx(idx_b, isem_b):
            pltpu.make_async_copy(idx_hbm.at[0, 0], idx_b, isem_b).wait()

        def start_gathers(idx_b, rows_b, gsem_b):
            pltpu.async_copy(table_hbm.at[idx_b], rows_b, gsem_b)

        def wait_gathers(idx_b, rows_b, gsem_b):
            pltpu.make_async_copy(table_hbm.at[idx_b], rows_b, gsem_b).wait()

        def wait_tr_store(tr_b, tsem_b):
            for dt in range(2):
                pltpu.make_async_copy(
                    tr_b.at[pl.ds(dt * 8, 8), pl.ds(0, 128)],
                    out_hbm.at[0, dt, 0], tsem_b).wait()

        def transpose_store_block(t, rows_b):
            lo = t // BT
            bt = t - lo * BT

            def tpair(q, carry):
                for half in range(2):
                    tr_b, tsem_b = (trA, tsemA) if half == 0 else (trB, tsemB)

                    @pl.when(q > 0)
                    def _():
                        wait_tr_store(tr_b, tsem_b)

                    li = 2 * q + half
                    for jj in range(128):
                        v = rows_b[li * 128 + jj, :]
                        col = jnp.full((16,), jj, jnp.int32)
                        plsc.store_scatter(tr_b, [rows16, col], v)
                    l_out = lo * 8 + li
                    for dt in range(2):
                        pltpu.async_copy(
                            tr_b.at[pl.ds(dt * 8, 8), pl.ds(0, 128)],
                            out_hbm.at[l_out, dt, bt], tsem_b)
                return carry

            lax.fori_loop(0, 4, tpair, 0)
            wait_tr_store(trA, tsemA)
            wait_tr_store(trB, tsemB)

        start_idx(t0, idxA, isemA)
        wait_idx(idxA, isemA)
        start_gathers(idxA, rowsA, gsemA)
        start_idx(t0 + 1, idxB, isemB)

        def pair(p, carry):
            t = t0 + 2 * p
            wait_gathers(idxA, rowsA, gsemA)
            wait_idx(idxB, isemB)
            start_gathers(idxB, rowsB, gsemB)
            start_idx(t + 2, idxA, isemA)
            transpose_store_block(t, rowsA)
            wait_gathers(idxB, rowsB, gsemB)
            wait_idx(idxA, isemA)
            start_gathers(idxA, rowsA, gsemA)
            start_idx(t + 3, idxB, isemB)
            transpose_store_block(t + 1, rowsB)
            return carry

        lax.fori_loop(0, PER_W // 2, pair, 0)

        wait_gathers(idxA, rowsA, gsemA)
        wait_idx(idxB, isemB)

      pl.run_scoped(
          scoped,
          pltpu.VMEM((1024, DIM), jnp.float32),
          pltpu.VMEM((1024, DIM), jnp.float32),
          pltpu.VMEM((16, 129), jnp.float32),
          pltpu.VMEM((16, 129), jnp.float32),
      )

    return k(table, idx4)


def kernel(conditions, table):
    idx4 = conditions.reshape(128, 128, 25, 8).transpose(2, 0, 3, 1)
    idx4 = idx4.reshape(25, 128, 1024)
    out5 = _sc_embed(idx4, table)
    out5 = out5.reshape(L, 2, BT, 8, 128)
    return out5.transpose(2, 4, 0, 1, 3).reshape(B, L, DIM)
